# baseline (speedup 1.0000x reference)
import functools

import jax
import jax.numpy as jnp
from jax import lax
from jax.experimental import pallas as pl
from jax.experimental.pallas import tpu as pltpu
from jax.experimental.pallas import tpu_sc as plsc

D = 64
SCALE = 8.0
B_TOT = 4096 * 200

_info = plsc.get_sparse_core_info()
NC, NS, L = _info.num_cores, _info.num_subcores, _info.num_lanes
NW = NC * NS
PER_W = B_TOT // NW
CHUNK = 512
STEPS = PER_W // CHUNK

_mesh = plsc.VectorSubcoreMesh(core_axis_name="c", subcore_axis_name="s")


@functools.partial(
    pl.kernel,
    mesh=_mesh,
    compiler_params=pltpu.CompilerParams(use_tc_tiling_on_sc=False),
    out_type=jax.ShapeDtypeStruct((B_TOT, D), jnp.float32),
    scratch_types=[
        pltpu.VMEM((CHUNK,), jnp.int32),
        pltpu.VMEM((CHUNK, D), jnp.float32),
        pltpu.SemaphoreType.DMA,
    ],
)
def _gather_scale(x_hbm, table_hbm, out_hbm, idx_v, rows_v, sem):
    wid = lax.axis_index("s") * NC + lax.axis_index("c")
    base_w = wid * PER_W

    def step(s, carry):
        base = base_w + s * CHUNK
        pltpu.sync_copy(x_hbm.at[pl.ds(base, CHUNK)], idx_v)
        pltpu.async_copy(table_hbm.at[idx_v], rows_v, sem).wait()

        def scale_row(r, c2):
            for c in range(D // L):
                sl = pl.ds(c * L, L)
                rows_v[r, sl] = rows_v[r, sl] * SCALE
            return c2

        lax.fori_loop(0, CHUNK, scale_row, 0)
        pltpu.sync_copy(rows_v, out_hbm.at[pl.ds(base, CHUNK)])
        return carry

    lax.fori_loop(0, STEPS, step, 0)


def kernel(x, input_embedding_table):
    out = _gather_scale(x.reshape(-1), input_embedding_table)
    return out.reshape(x.shape[0], x.shape[1], D)


# speedup vs baseline: 1.1372x; 1.1372x over previous
import functools

import jax
import jax.numpy as jnp
from jax import lax
from jax.experimental import pallas as pl
from jax.experimental.pallas import tpu as pltpu
from jax.experimental.pallas import tpu_sc as plsc

D = 64
SCALE = 8.0
B_TOT = 4096 * 200

_info = plsc.get_sparse_core_info()
NC, NS, L = _info.num_cores, _info.num_subcores, _info.num_lanes
NW = NC * NS
PER_W = B_TOT // NW
CHUNK = 512
STEPS = PER_W // CHUNK
RU = 4

_mesh = plsc.VectorSubcoreMesh(core_axis_name="c", subcore_axis_name="s")


@functools.partial(
    pl.kernel,
    mesh=_mesh,
    compiler_params=pltpu.CompilerParams(use_tc_tiling_on_sc=False),
    out_type=jax.ShapeDtypeStruct((B_TOT, D), jnp.float32),
    scratch_types=[
        pltpu.VMEM((CHUNK,), jnp.int32),
        pltpu.VMEM((CHUNK,), jnp.int32),
        pltpu.VMEM((CHUNK, D), jnp.float32),
        pltpu.VMEM((CHUNK, D), jnp.float32),
        pltpu.SemaphoreType.DMA,
        pltpu.SemaphoreType.DMA,
        pltpu.SemaphoreType.DMA,
        pltpu.SemaphoreType.DMA,
    ],
)
def _gather_scale(x_hbm, table_hbm, out_hbm,
                  idx_a, idx_b, rows_a, rows_b, ga, gb, sta, stb):
    wid = lax.axis_index("s") * NC + lax.axis_index("c")
    base_w = wid * PER_W
    idx = (idx_a, idx_b)
    rows = (rows_a, rows_b)
    gsem = (ga, gb)
    ssem = (sta, stb)

    def xs(p):
        return x_hbm.at[pl.ds(base_w + p * CHUNK, CHUNK)]

    def os(p):
        return out_hbm.at[pl.ds(base_w + p * CHUNK, CHUNK)]

    def scale(rv):
        def body(r2, c):
            r0 = r2 * RU
            for u in range(RU):
                for cc in range(D // L):
                    sl = pl.ds(cc * L, L)
                    rv[r0 + u, sl] = rv[r0 + u, sl] * SCALE
            return c
        lax.fori_loop(0, CHUNK // RU, body, 0)


    pltpu.sync_copy(xs(0), idx[0])
    pltpu.async_copy(table_hbm.at[idx[0]], rows[0], gsem[0])
    pltpu.sync_copy(xs(1), idx[1])
    pltpu.make_async_copy(table_hbm.at[idx[0]], rows[0], gsem[0]).wait()
    pltpu.async_copy(table_hbm.at[idx[1]], rows[1], gsem[1])
    scale(rows[0])
    pltpu.async_copy(rows[0], os(0), ssem[0])

    def pair(k, c):
        for j in (0, 1):
            b = (1 + j) % 2
            s = 2 * k + 1 + j
            pltpu.sync_copy(xs(s + 1), idx[1 - b])
            pltpu.make_async_copy(table_hbm.at[idx[b]], rows[b], gsem[b]).wait()
            pltpu.make_async_copy(rows[1 - b], os(0), ssem[1 - b]).wait()
            pltpu.async_copy(table_hbm.at[idx[1 - b]], rows[1 - b], gsem[1 - b])
            scale(rows[b])
            pltpu.async_copy(rows[b], os(s), ssem[b])
        return c

    lax.fori_loop(0, (STEPS - 2) // 2, pair, 0)

    pltpu.make_async_copy(table_hbm.at[idx[1]], rows[1], gsem[1]).wait()
    pltpu.make_async_copy(rows[0], os(0), ssem[0]).wait()
    scale(rows[1])
    pltpu.async_copy(rows[1], os(STEPS - 1), ssem[1])
    pltpu.make_async_copy(rows[1], os(0), ssem[1]).wait()


def kernel(x, input_embedding_table):
    out = _gather_scale(x.reshape(-1), input_embedding_table)
    return out.reshape(x.shape[0], x.shape[1], D)
